# baseline (speedup 1.0000x reference)
import jax
import jax.numpy as jnp
from jax.experimental import pallas as pl

N = 10000
HIDDEN = 64
NUM_GRAPHS = 64


def _gat_conv(x, src, dst, W, a_src, a_dst, bias, heads, out_ch, concat):
    n = x.shape[0]
    loop = jnp.arange(n, dtype=src.dtype)
    src = jnp.concatenate([src, loop])
    dst = jnp.concatenate([dst, loop])
    h = (x @ W).reshape(n, heads, out_ch)
    alpha_src = jnp.sum(h * a_src[None, :, :], axis=-1)
    alpha_dst = jnp.sum(h * a_dst[None, :, :], axis=-1)
    e = jax.nn.leaky_relu(alpha_src[src] + alpha_dst[dst], negative_slope=0.2)
    m = jax.ops.segment_max(e, dst, num_segments=n)
    m = jnp.where(jnp.isfinite(m), m, 0.0)
    ex = jnp.exp(e - m[dst])
    denom = jax.ops.segment_sum(ex, dst, num_segments=n)
    attn = ex / (denom[dst] + 1e-16)
    out = jax.ops.segment_sum(h[src] * attn[:, :, None], dst, num_segments=n)
    if concat:
        out = out.reshape(n, heads * out_ch)
    else:
        out = out.mean(axis=1)
    return out + bias


def _gcn_conv(x, src, dst, W, bias):
    n = x.shape[0]
    loop = jnp.arange(n, dtype=src.dtype)
    src = jnp.concatenate([src, loop])
    dst = jnp.concatenate([dst, loop])
    h = x @ W
    deg = jax.ops.segment_sum(jnp.ones(dst.shape, h.dtype), dst, num_segments=n)
    dinv = jnp.where(deg > 0, deg ** -0.5, 0.0)
    norm = dinv[src] * dinv[dst]
    out = jax.ops.segment_sum(h[src] * norm[:, None], dst, num_segments=n)
    return out + bias


def _head_kernel(p_ref, wc1_ref, bc1_ref, wc2_ref, bc2_ref, wp1_ref, bp1_ref,
                 wp2_ref, bp2_ref, situ_ref, path_ref):
    p = p_ref[...]
    h1 = jnp.maximum(p @ wc1_ref[...] + bc1_ref[...], 0.0)
    situ_ref[...] = h1 @ wc2_ref[...] + bc2_ref[...]
    h2 = jnp.maximum(p @ wp1_ref[...] + bp1_ref[...], 0.0)
    path_ref[...] = h2 @ wp2_ref[...] + bp2_ref[...]


def kernel(x, edge_index, edge_attr, batch, W1, a_src1, a_dst1, b1, W2,
           a_src2, a_dst2, b2, Wg, bg, Wf, bf, Wc1, bc1, Wc2, bc2, Wp1, bp1,
           Wp2, bp2):
    src, dst = edge_index[0], edge_index[1]
    g1 = jax.nn.elu(_gat_conv(x, src, dst, W1, a_src1, a_dst1, b1, 4, HIDDEN, True))
    g2 = jax.nn.elu(_gat_conv(g1, src, dst, W2, a_src2, a_dst2, b2, 1, HIDDEN, False))
    gc = jax.nn.relu(_gcn_conv(x, src, dst, Wg, bg))
    fused = jax.nn.relu(jnp.concatenate([g2, gc], axis=-1) @ Wf + bf)
    s = jax.ops.segment_sum(fused, batch, num_segments=NUM_GRAPHS)
    cnt = jax.ops.segment_sum(jnp.ones((fused.shape[0],), fused.dtype), batch,
                              num_segments=NUM_GRAPHS)
    pooled = s / jnp.maximum(cnt, 1.0)[:, None]
    situ, path_mod = pl.pallas_call(
        _head_kernel,
        out_shape=(jax.ShapeDtypeStruct((NUM_GRAPHS, 5), jnp.float32),
                   jax.ShapeDtypeStruct((NUM_GRAPHS, 5), jnp.float32)),
    )(pooled, Wc1, bc1, Wc2, bc2, Wp1, bp1, Wp2, bp2)
    return (situ, path_mod)


# speedup vs baseline: 27.7764x; 27.7764x over previous
import jax
import jax.numpy as jnp
from jax import lax
from jax.experimental import pallas as pl
from jax.experimental.pallas import tpu as pltpu
from jax.experimental.pallas import tpu_sc as plsc

N = 10000
HIDDEN = 64
NUM_GRAPHS = 64

_NC, _NS = 2, 16
_E = 320000
_C = 80
_NCHUNK = _E // _C
_NPAD = 10240
_RPT = _NPAD // _NS
_TW1 = 176
_TW2 = 80
_RB = 1024


def _leaky(v):
    return jnp.where(v > 0, v, 0.2 * v)


def _elu(v):
    return jnp.where(v > 0, v, jnp.exp(v) - 1.0)


def _deg_body(dst_hbm, zd_hbm, deg_out, dstv, onesv, deg_sh):
    c = lax.axis_index("c")
    s = lax.axis_index("s")
    wid = s * _NC + c
    cpw = _NCHUNK // (_NC * _NS)

    for i in range(_C // 16):
        onesv[pl.ds(i * 16, 16)] = jnp.ones((16,), jnp.float32)
    pltpu.sync_copy(zd_hbm, deg_sh.at[pl.ds(s * _RPT, _RPT)])
    plsc.subcore_barrier()

    def chunk(k, carry):
        base = (wid * cpw + k) * _C
        pltpu.sync_copy(dst_hbm.at[pl.ds(base, _C)], dstv)
        pltpu.sync_copy(onesv, deg_sh.at[dstv], add=True)
        return carry

    lax.fori_loop(0, cpw, chunk, 0)
    plsc.subcore_barrier()
    pltpu.sync_copy(deg_sh.at[pl.ds(s * _RPT, _RPT)],
                    deg_out.at[pl.ds(c * _NPAD + s * _RPT, _RPT)])


def _sc_degree(dst, zd):
    return pl.kernel(
        _deg_body,
        out_type=jax.ShapeDtypeStruct((_NC * _NPAD,), jnp.float32),
        mesh=plsc.VectorSubcoreMesh(core_axis_name="c", subcore_axis_name="s"),
        compiler_params=pltpu.CompilerParams(use_tc_tiling_on_sc=False),
        scratch_types=[
            pltpu.VMEM((_C,), jnp.int32),
            pltpu.VMEM((_C,), jnp.float32),
            pltpu.VMEM_SHARED((_NPAD,), jnp.float32),
        ],
    )(dst, zd)


def _pass1_body(src_hbm, dst_hbm, t0_hbm, t1_hbm, ad0_hbm, ad1_hbm, z1_hbm,
                p0_out, p1_out, srcv, dstv, rows, adv, acc_sh):
    c = lax.axis_index("c")
    s = lax.axis_index("s")
    cpt = _NCHUNK // _NS

    pltpu.sync_copy(z1_hbm, acc_sh.at[pl.ds(s * _RPT, _RPT)])
    plsc.subcore_barrier()

    def chunk(k, carry):
        base = (s * cpt + k) * _C
        pltpu.sync_copy(src_hbm.at[pl.ds(base, _C)], srcv)
        pltpu.sync_copy(dst_hbm.at[pl.ds(base, _C)], dstv)

        @pl.when(c == 0)
        def _():
            pltpu.sync_copy(t0_hbm.at[srcv], rows)
            pltpu.sync_copy(ad0_hbm.at[dstv], adv)

        @pl.when(c == 1)
        def _():
            pltpu.sync_copy(t1_hbm.at[srcv], rows)
            pltpu.sync_copy(ad1_hbm.at[dstv], adv)

        def edge(e, cc):
            wpk = jnp.exp(_leaky(rows[e, pl.ds(128, 16)] + adv[e, pl.ds(0, 16)]))
            rows[e, pl.ds(128, 16)] = wpk
            for j in range(8):
                rows[e, pl.ds(j * 16, 16)] = rows[e, pl.ds(j * 16, 16)] * wpk
            return cc

        lax.fori_loop(0, _C, edge, 0)
        pltpu.sync_copy(rows, acc_sh.at[dstv], add=True)
        return carry

    lax.fori_loop(0, cpt, chunk, 0)
    plsc.subcore_barrier()

    @pl.when(c == 0)
    def _():
        pltpu.sync_copy(acc_sh.at[pl.ds(s * _RPT, _RPT)],
                        p0_out.at[pl.ds(s * _RPT, _RPT)])

    @pl.when(c == 1)
    def _():
        pltpu.sync_copy(acc_sh.at[pl.ds(s * _RPT, _RPT)],
                        p1_out.at[pl.ds(s * _RPT, _RPT)])


def _sc_pass1(src, dst, t0, t1, ad0, ad1, z1):
    return pl.kernel(
        _pass1_body,
        out_type=(jax.ShapeDtypeStruct((_NPAD, _TW1), jnp.float32),
                  jax.ShapeDtypeStruct((_NPAD, _TW1), jnp.float32)),
        mesh=plsc.VectorSubcoreMesh(core_axis_name="c", subcore_axis_name="s"),
        compiler_params=pltpu.CompilerParams(use_tc_tiling_on_sc=False),
        scratch_types=[
            pltpu.VMEM((_C,), jnp.int32),
            pltpu.VMEM((_C,), jnp.int32),
            pltpu.VMEM((_C, _TW1), jnp.float32),
            pltpu.VMEM((_C, 16), jnp.float32),
            pltpu.VMEM_SHARED((_NPAD, _TW1), jnp.float32),
        ],
    )(src, dst, t0, t1, ad0, ad1, z1)


def _pass2_body(src_hbm, dst_hbm, t2_hbm, ad2_hbm, z2_hbm, p2_out,
                srcv, dstv, rows, adv, acc_sh):
    c = lax.axis_index("c")
    s = lax.axis_index("s")
    wid = s * _NC + c
    cpw = _NCHUNK // (_NC * _NS)

    pltpu.sync_copy(z2_hbm, acc_sh.at[pl.ds(s * _RPT, _RPT)])
    plsc.subcore_barrier()

    def chunk(k, carry):
        base = (wid * cpw + k) * _C
        pltpu.sync_copy(src_hbm.at[pl.ds(base, _C)], srcv)
        pltpu.sync_copy(dst_hbm.at[pl.ds(base, _C)], dstv)
        pltpu.sync_copy(t2_hbm.at[srcv], rows)
        pltpu.sync_copy(ad2_hbm.at[dstv], adv)

        def edge(e, cc):
            wv = jnp.exp(_leaky(rows[e, pl.ds(64, 16)] + adv[e, pl.ds(0, 16)]))
            rows[e, pl.ds(64, 16)] = wv
            for j in range(4):
                rows[e, pl.ds(j * 16, 16)] = rows[e, pl.ds(j * 16, 16)] * wv
            return cc

        lax.fori_loop(0, _C, edge, 0)
        pltpu.sync_copy(rows, acc_sh.at[dstv], add=True)
        return carry

    lax.fori_loop(0, cpw, chunk, 0)
    plsc.subcore_barrier()
    pltpu.sync_copy(acc_sh.at[pl.ds(s * _RPT, _RPT)],
                    p2_out.at[pl.ds(c * _NPAD + s * _RPT, _RPT)])


def _sc_pass2(src, dst, t2, ad2, z2):
    return pl.kernel(
        _pass2_body,
        out_type=jax.ShapeDtypeStruct((_NC * _NPAD, _TW2), jnp.float32),
        mesh=plsc.VectorSubcoreMesh(core_axis_name="c", subcore_axis_name="s"),
        compiler_params=pltpu.CompilerParams(use_tc_tiling_on_sc=False),
        scratch_types=[
            pltpu.VMEM((_C,), jnp.int32),
            pltpu.VMEM((_C,), jnp.int32),
            pltpu.VMEM((_C, _TW2), jnp.float32),
            pltpu.VMEM((_C, 16), jnp.float32),
            pltpu.VMEM_SHARED((_NPAD, _TW2), jnp.float32),
        ],
    )(src, dst, t2, ad2, z2)


def _rep8(col):
    return jnp.broadcast_to(col, (col.shape[0], 8))


def _interleave(ha, hb):
    parts = []
    for j in range(8):
        parts.append(ha[:, 8 * j:8 * j + 8])
        parts.append(hb[:, 8 * j:8 * j + 8])
    return jnp.concatenate(parts, axis=1)


def _pack2(a, b):
    return jnp.concatenate([_rep8(a), _rep8(b)], axis=1)


def _prep_kernel(x_ref, w1_ref, as_ref, adm_ref, wg_ref, degp_ref,
                 t0_ref, t1_ref, ad0_ref, ad1_ref, wself_ref, dinv_ref):
    xb = x_ref[...]
    h1 = xb @ w1_ref[...]
    asb = h1 @ as_ref[...]
    adb = h1 @ adm_ref[...]
    deg = degp_ref[0, :] + degp_ref[1, :] + 1.0
    dinv = lax.rsqrt(deg)
    hg = (xb @ wg_ref[...]) * dinv[:, None]
    t0_ref[...] = jnp.concatenate(
        [_interleave(h1[:, 0:64], h1[:, 64:128]),
         _pack2(asb[:, 0:1], asb[:, 1:2]), hg[:, 0:32]], axis=1)
    t1_ref[...] = jnp.concatenate(
        [_interleave(h1[:, 128:192], h1[:, 192:256]),
         _pack2(asb[:, 2:3], asb[:, 3:4]), hg[:, 32:64]], axis=1)
    ad0_ref[...] = jnp.concatenate([_pack2(adb[:, 0:1], adb[:, 1:2])], axis=1)
    ad1_ref[...] = jnp.concatenate([_pack2(adb[:, 2:3], adb[:, 3:4])], axis=1)
    wself_ref[...] = jnp.exp(_leaky(asb + adb))
    dinv_ref[...] = dinv


def _tc_prep(x, W1, A_s, A_d, Wg, deg2):
    grid = _NPAD // _RB
    return pl.pallas_call(
        _prep_kernel,
        grid=(grid,),
        in_specs=[
            pl.BlockSpec((_RB, 128), lambda i: (i, 0)),
            pl.BlockSpec((128, 256), lambda i: (0, 0)),
            pl.BlockSpec((256, 4), lambda i: (0, 0)),
            pl.BlockSpec((256, 4), lambda i: (0, 0)),
            pl.BlockSpec((128, 64), lambda i: (0, 0)),
            pl.BlockSpec((_NC, _RB), lambda i: (0, i)),
        ],
        out_specs=[
            pl.BlockSpec((_RB, _TW1), lambda i: (i, 0)),
            pl.BlockSpec((_RB, _TW1), lambda i: (i, 0)),
            pl.BlockSpec((_RB, 16), lambda i: (i, 0)),
            pl.BlockSpec((_RB, 16), lambda i: (i, 0)),
            pl.BlockSpec((_RB, 4), lambda i: (i, 0)),
            pl.BlockSpec((_RB,), lambda i: (i,)),
        ],
        out_shape=[
            jax.ShapeDtypeStruct((_NPAD, _TW1), jnp.float32),
            jax.ShapeDtypeStruct((_NPAD, _TW1), jnp.float32),
            jax.ShapeDtypeStruct((_NPAD, 16), jnp.float32),
            jax.ShapeDtypeStruct((_NPAD, 16), jnp.float32),
            jax.ShapeDtypeStruct((_NPAD, 4), jnp.float32),
            jax.ShapeDtypeStruct((_NPAD,), jnp.float32),
        ],
    )(x, W1, A_s, A_d, Wg, deg2)


def _mid_kernel(p0_ref, p1_ref, t0_ref, t1_ref, wself_ref, dinv_ref,
                b1_ref, w2_ref, as2_ref, ad2_ref, bg_ref,
                t2_ref, adt_ref, gc_ref, wself2_ref):
    p0 = p0_ref[...]
    p1 = p1_ref[...]
    t0 = t0_ref[...]
    t1 = t1_ref[...]
    wself = wself_ref[...]
    b1 = b1_ref[...]
    g1_parts = []
    for c, (pc, tc) in enumerate(((p0, t0), (p1, t1))):
        denp = _pack2(pc[:, 128:129] + wself[:, 2 * c:2 * c + 1],
                      pc[:, 136:137] + wself[:, 2 * c + 1:2 * c + 2]) + 1e-16
        wsp = _pack2(wself[:, 2 * c:2 * c + 1], wself[:, 2 * c + 1:2 * c + 2])
        den = jnp.tile(denp, (1, 8))
        ws = jnp.tile(wsp, (1, 8))
        acc = pc[:, 0:128] + ws * tc[:, 0:128]
        g1_parts.append(_elu(acc / den + b1[None, 128 * c:128 * c + 128]))
    g1 = jnp.concatenate(g1_parts, axis=1)
    h2 = g1 @ w2_ref[...]
    as2 = h2 @ as2_ref[...]
    ad2 = h2 @ ad2_ref[...]
    t2_ref[...] = jnp.concatenate([h2, jnp.broadcast_to(as2, (as2.shape[0], 16))],
                                  axis=1)
    adt_ref[...] = jnp.broadcast_to(ad2, (ad2.shape[0], 16))
    accg = jnp.concatenate([p0[:, 144:176], p1[:, 144:176]], axis=1)
    hgs = jnp.concatenate([t0[:, 144:176], t1[:, 144:176]], axis=1)
    dinv = dinv_ref[...]
    gc_ref[...] = jnp.maximum(dinv[:, None] * (accg + hgs) + bg_ref[...][None, :], 0.0)
    wself2_ref[...] = jnp.exp(_leaky(as2[:, 0] + ad2[:, 0]))


def _tc_mid(p0, p1, t0, t1, wself1, dinv, b1, W2, a_s2, a_d2, bg):
    grid = _NPAD // _RB
    return pl.pallas_call(
        _mid_kernel,
        grid=(grid,),
        in_specs=[
            pl.BlockSpec((_RB, _TW1), lambda i: (i, 0)),
            pl.BlockSpec((_RB, _TW1), lambda i: (i, 0)),
            pl.BlockSpec((_RB, _TW1), lambda i: (i, 0)),
            pl.BlockSpec((_RB, _TW1), lambda i: (i, 0)),
            pl.BlockSpec((_RB, 4), lambda i: (i, 0)),
            pl.BlockSpec((_RB,), lambda i: (i,)),
            pl.BlockSpec((256,), lambda i: (0,)),
            pl.BlockSpec((256, 64), lambda i: (0, 0)),
            pl.BlockSpec((64, 1), lambda i: (0, 0)),
            pl.BlockSpec((64, 1), lambda i: (0, 0)),
            pl.BlockSpec((64,), lambda i: (0,)),
        ],
        out_specs=[
            pl.BlockSpec((_RB, _TW2), lambda i: (i, 0)),
            pl.BlockSpec((_RB, 16), lambda i: (i, 0)),
            pl.BlockSpec((_RB, 64), lambda i: (i, 0)),
            pl.BlockSpec((_RB,), lambda i: (i,)),
        ],
        out_shape=[
            jax.ShapeDtypeStruct((_NPAD, _TW2), jnp.float32),
            jax.ShapeDtypeStruct((_NPAD, 16), jnp.float32),
            jax.ShapeDtypeStruct((_NPAD, 64), jnp.float32),
            jax.ShapeDtypeStruct((_NPAD,), jnp.float32),
        ],
    )(p0, p1, t0, t1, wself1, dinv, b1, W2, a_s2, a_d2, bg)


def _fin_kernel(p2a_ref, p2b_ref, t2_ref, wself2_ref, b2_ref, gc_ref,
                wf_ref, bf_ref, batch_ref, wc1_ref, bc1_ref, wc2_ref, bc2_ref,
                wp1_ref, bp1_ref, wp2_ref, bp2_ref,
                situ_ref, path_ref, s_acc, cnt_acc):
    i = pl.program_id(0)
    nsteps = pl.num_programs(0)

    @pl.when(i == 0)
    def _():
        s_acc[...] = jnp.zeros_like(s_acc)
        cnt_acc[...] = jnp.zeros_like(cnt_acc)

    h2 = t2_ref[...][:, 0:64]
    wself2 = wself2_ref[...]
    acc2 = p2a_ref[...][:, 0:64] + p2b_ref[...][:, 0:64] + wself2[:, None] * h2
    den2 = p2a_ref[...][:, 64] + p2b_ref[...][:, 64] + wself2 + 1e-16
    g2 = _elu(acc2 / den2[:, None] + b2_ref[...][None, :])
    wf = wf_ref[...]
    fused = jnp.maximum(g2 @ wf[0:64, :] + gc_ref[...] @ wf[64:128, :]
                        + bf_ref[...][None, :], 0.0)
    rb = fused.shape[0]
    rowid = i * rb + lax.broadcasted_iota(jnp.int32, (rb, 1), 0)[:, 0]
    valid = rowid < N
    batch = batch_ref[...]
    gids = lax.broadcasted_iota(jnp.int32, (rb, NUM_GRAPHS), 1)
    onehot = jnp.where((batch[:, None] == gids) & valid[:, None], 1.0, 0.0)
    s_acc[...] += jax.lax.dot_general(onehot, fused, (((0,), (0,)), ((), ())))
    cnt_acc[...] += jnp.sum(onehot, axis=0)

    @pl.when(i == nsteps - 1)
    def _():
        pooled = s_acc[...] / jnp.maximum(cnt_acc[...], 1.0)[:, None]
        hh1 = jnp.maximum(pooled @ wc1_ref[...] + bc1_ref[...][None, :], 0.0)
        situ_ref[...] = hh1 @ wc2_ref[...] + bc2_ref[...][None, :]
        hh2 = jnp.maximum(pooled @ wp1_ref[...] + bp1_ref[...][None, :], 0.0)
        path_ref[...] = hh2 @ wp2_ref[...] + bp2_ref[...][None, :]


def _tc_final(p2, t2e, wself2, b2, gc, Wf, bf, batch_pad,
              Wc1, bc1, Wc2, bc2, Wp1, bp1, Wp2, bp2):
    grid = _NPAD // _RB
    return pl.pallas_call(
        _fin_kernel,
        grid=(grid,),
        in_specs=[
            pl.BlockSpec((_RB, _TW2), lambda i: (i, 0)),
            pl.BlockSpec((_RB, _TW2), lambda i: (i + _NPAD // _RB, 0)),
            pl.BlockSpec((_RB, _TW2), lambda i: (i, 0)),
            pl.BlockSpec((_RB,), lambda i: (i,)),
            pl.BlockSpec((64,), lambda i: (0,)),
            pl.BlockSpec((_RB, 64), lambda i: (i, 0)),
            pl.BlockSpec((128, 64), lambda i: (0, 0)),
            pl.BlockSpec((64,), lambda i: (0,)),
            pl.BlockSpec((_RB,), lambda i: (i,)),
            pl.BlockSpec((64, 32), lambda i: (0, 0)),
            pl.BlockSpec((32,), lambda i: (0,)),
            pl.BlockSpec((32, 5), lambda i: (0, 0)),
            pl.BlockSpec((5,), lambda i: (0,)),
            pl.BlockSpec((64, 32), lambda i: (0, 0)),
            pl.BlockSpec((32,), lambda i: (0,)),
            pl.BlockSpec((32, 5), lambda i: (0, 0)),
            pl.BlockSpec((5,), lambda i: (0,)),
        ],
        out_specs=[
            pl.BlockSpec((NUM_GRAPHS, 5), lambda i: (0, 0)),
            pl.BlockSpec((NUM_GRAPHS, 5), lambda i: (0, 0)),
        ],
        out_shape=[
            jax.ShapeDtypeStruct((NUM_GRAPHS, 5), jnp.float32),
            jax.ShapeDtypeStruct((NUM_GRAPHS, 5), jnp.float32),
        ],
        scratch_shapes=[
            pltpu.VMEM((NUM_GRAPHS, 64), jnp.float32),
            pltpu.VMEM((NUM_GRAPHS,), jnp.float32),
        ],
    )(p2, p2, t2e, wself2, b2, gc, Wf, bf, batch_pad,
      Wc1, bc1, Wc2, bc2, Wp1, bp1, Wp2, bp2)


def kernel(x, edge_index, edge_attr, batch, W1, a_src1, a_dst1, b1, W2,
           a_src2, a_dst2, b2, Wg, bg, Wf, bf, Wc1, bc1, Wc2, bc2, Wp1, bp1,
           Wp2, bp2):
    src = edge_index[0]
    dst = edge_index[1]
    eye4 = jnp.eye(4, dtype=jnp.float32)
    A_s = (eye4[:, None, :] * a_src1[:, :, None]).reshape(256, 4)
    A_d = (eye4[:, None, :] * a_dst1[:, :, None]).reshape(256, 4)
    perm = [64 * (2 * (k // 128) + ((k % 16) >= 8)) + 8 * ((k % 128) // 16)
            + (k % 8) for k in range(256)]
    perm = jnp.array(perm, dtype=jnp.int32)
    W2p = W2[perm, :]
    b1p = b1[perm]
    x_pad = jnp.concatenate(
        [x, jnp.zeros((_NPAD - N, x.shape[1]), jnp.float32)])
    zd = jnp.zeros((_RPT,), jnp.float32)
    z1 = jnp.zeros((_RPT, _TW1), jnp.float32)
    z2 = jnp.zeros((_RPT, _TW2), jnp.float32)
    batch_pad = jnp.concatenate([batch, jnp.full((_NPAD - N,), -1, batch.dtype)])

    deg = _sc_degree(dst, zd).reshape(_NC, _NPAD)
    t0, t1, ad0, ad1, wself1, dinv = _tc_prep(x_pad, W1, A_s, A_d, Wg, deg)
    p0, p1 = _sc_pass1(src, dst, t0, t1, ad0, ad1, z1)
    t2e, ad2t, gc, wself2 = _tc_mid(p0, p1, t0, t1, wself1, dinv, b1p, W2p,
                                    a_src2.reshape(64, 1), a_dst2.reshape(64, 1),
                                    bg)
    p2 = _sc_pass2(src, dst, t2e, ad2t, z2)
    situ, path_mod = _tc_final(p2, t2e, wself2, b2, gc, Wf, bf, batch_pad,
                               Wc1, bc1, Wc2, bc2, Wp1, bp1, Wp2, bp2)
    return (situ, path_mod)
